# baseline (speedup 1.0000x reference)
import jax
import jax.numpy as jnp
from jax import lax
from jax.experimental import pallas as pl
from jax.experimental.pallas import tpu as pltpu
from jax.experimental.pallas import tpu_sc as plsc

_FACTOR = 64
_BATCH = 16384
_NC = 2
_NS = 16
_L = 16
_NW = _NC * _NS
_BPW = _BATCH // _NW
_NJ = _BPW // 128


def _pmf_body(wu_hbm, wi_hbm, user_hbm, item_hbm, out_hbm,
              idx_u, idx_i, rows_u, rows_i, out_v, sem):
    wid = lax.axis_index("s") * _NC + lax.axis_index("c")
    base = wid * _NJ

    pltpu.sync_copy(user_hbm.at[pl.ds(base, _NJ)], idx_u)
    pltpu.sync_copy(item_hbm.at[pl.ds(base, _NJ)], idx_i)

    copies = []
    for j in range(_NJ):
        copies.append(pltpu.async_copy(
            wu_hbm.at[idx_u.at[j]], rows_u.at[pl.ds(j * 128, 128)], sem))
        copies.append(pltpu.async_copy(
            wi_hbm.at[idx_i.at[j]], rows_i.at[pl.ds(j * 128, 128)], sem))
    for c in copies:
        c.wait()

    lane = lax.iota(jnp.int32, _L)

    def group(g, _):
        row = lane + g * _L
        acc0 = jnp.zeros((_L,), jnp.float32)
        acc1 = jnp.zeros((_L,), jnp.float32)
        col = jnp.zeros((_L,), jnp.int32)
        for c in range(0, _FACTOR, 2):
            u0 = plsc.load_gather(rows_u, [row, col])
            v0 = plsc.load_gather(rows_i, [row, col])
            acc0 = acc0 + u0 * v0
            col1 = col + 1
            u1 = plsc.load_gather(rows_u, [row, col1])
            v1 = plsc.load_gather(rows_i, [row, col1])
            acc1 = acc1 + u1 * v1
            col = col + 2
        out_v[pl.ds(g * _L, _L)] = acc0 + acc1
        return 0

    lax.fori_loop(0, _BPW // _L, group, 0)

    pltpu.sync_copy(out_v, out_hbm.at[pl.ds(wid * _BPW, _BPW)])


def kernel(user, item, W_user, W_item):
    user = user.astype(jnp.int32).reshape(_NW * _NJ, 128)
    item = item.astype(jnp.int32).reshape(_NW * _NJ, 128)
    mesh = plsc.VectorSubcoreMesh(core_axis_name="c", subcore_axis_name="s")
    run = pl.kernel(
        _pmf_body,
        out_type=jax.ShapeDtypeStruct((_BATCH,), jnp.float32),
        mesh=mesh,
        compiler_params=pltpu.CompilerParams(
            needs_layout_passes=False, use_tc_tiling_on_sc=False),
        scratch_types=[
            pltpu.VMEM((_NJ, 128), jnp.int32),
            pltpu.VMEM((_NJ, 128), jnp.int32),
            pltpu.VMEM((_BPW, _FACTOR), jnp.float32),
            pltpu.VMEM((_BPW, _FACTOR), jnp.float32),
            pltpu.VMEM((_BPW,), jnp.float32),
            pltpu.SemaphoreType.DMA,
        ],
    )
    return run(W_user, W_item, user, item)
